# baseline (speedup 1.0000x reference)
import functools
import jax
import jax.numpy as jnp
from jax import lax
from jax.experimental import pallas as pl
from jax.experimental.pallas import tpu as pltpu, tpu_sc as plsc

V_DIM = 32
NC, NS = 2, 16
NW = NC * NS


NBUF = 4


@functools.lru_cache(maxsize=None)
def _make_sc_kernel(N, C, K, nchunk):
    per_w = N // NW
    S = C * K
    nsuper = per_w // S
    assert nsuper % NBUF == 0
    mesh = plsc.VectorSubcoreMesh(core_axis_name="c", subcore_axis_name="s")

    @functools.partial(
        pl.kernel,
        out_type=jax.ShapeDtypeStruct((N, V_DIM), jnp.float32),
        mesh=mesh,
        scratch_types=[
            pltpu.VMEM((nchunk, C), jnp.int32),
            pltpu.VMEM((nchunk, C), jnp.int32),
            pltpu.VMEM((nchunk, C), jnp.int32),
            pltpu.VMEM((NBUF, S, V_DIM), jnp.float32),
            [pltpu.SemaphoreType.DMA] * NBUF,
            [pltpu.SemaphoreType.DMA] * NBUF,
            [pltpu.SemaphoreType.DMA] * NBUF,
        ],
        compiler_params=pltpu.CompilerParams(use_tc_tiling_on_sc=False),
    )
    def k(vidx_hbm, pos_hbm, deg_hbm, Wv, Wp, Wd, out_hbm,
          iv, ip, idg, rows, sg, sa, sw):
        wid = lax.axis_index("s") * NC + lax.axis_index("c")
        base = wid * per_w
        pltpu.sync_copy(vidx_hbm.at[wid], iv)
        pltpu.sync_copy(pos_hbm.at[wid], ip)
        pltpu.sync_copy(deg_hbm.at[wid], idg)

        def fire_base(s, p):
            buf = rows.at[p]
            for t in range(K):
                pltpu.async_copy(
                    Wv.at[iv.at[s * K + t]], buf.at[pl.ds(t * C, C)], sg[p])

        def wait_base(s, p):
            for t in range(K):
                pltpu.make_async_copy(
                    Wv.at[iv.at[s * K + t]],
                    rows.at[p].at[pl.ds(t * C, C)], sg[p]).wait()

        def step(s, u):
            p = u % NBUF
            p2 = (u + 2) % NBUF
            buf = rows.at[p]
            wait_base(s, p)
            for t in range(K):
                dst = buf.at[pl.ds(t * C, C)]
                pltpu.async_copy(Wp.at[ip.at[s * K + t]], dst, sa[p], add=True)
                pltpu.async_copy(Wd.at[idg.at[s * K + t]], dst, sa[p], add=True)
            @pl.when((s >= 2) & (s + 2 < nsuper))
            def _():
                pltpu.make_async_copy(
                    rows.at[p2], out_hbm.at[pl.ds(base, S)], sw[p2]).wait()

            @pl.when(s + 2 < nsuper)
            def _():
                fire_base(s + 2, p2)
            for t in range(K):
                dst = buf.at[pl.ds(t * C, C)]
                pltpu.make_async_copy(
                    Wp.at[ip.at[s * K + t]], dst, sa[p]).wait()
                pltpu.make_async_copy(
                    Wd.at[idg.at[s * K + t]], dst, sa[p]).wait()
            pltpu.async_copy(buf, out_hbm.at[pl.ds(base + s * S, S)], sw[p])

        fire_base(0, 0)
        fire_base(1, 1)

        def round_(g, carry):
            for u in range(NBUF):
                step(g * NBUF + u, u)
            return carry

        lax.fori_loop(0, nsuper // NBUF, round_, 0)
        for p in range(NBUF):
            pltpu.make_async_copy(
                rows.at[p], out_hbm.at[pl.ds(base, S)], sw[p]).wait()

    return k


def kernel(vidx, pos, deg, W_vidx, W_pos, W_deg):
    B, L = vidx.shape
    N = B * L
    C = 128
    nchunk = N // (NW * C)
    iv = vidx.reshape(NW, nchunk, C).astype(jnp.int32)
    ip = pos.reshape(NW, nchunk, C).astype(jnp.int32)
    idg = deg.reshape(NW, nchunk, C).astype(jnp.int32)
    out = _make_sc_kernel(N, C, 2, nchunk)(iv, ip, idg, W_vidx, W_pos, W_deg)
    return out.reshape(B, L, V_DIM)


# speedup vs baseline: 1.0103x; 1.0103x over previous
import functools
import jax
import jax.numpy as jnp
from jax import lax
from jax.experimental import pallas as pl
from jax.experimental.pallas import tpu as pltpu, tpu_sc as plsc

V_DIM = 32
NC, NS = 2, 16
NW = NC * NS
NBUF = 2


@functools.lru_cache(maxsize=None)
def _make_sc_kernel(N, C, K, nchunk, DEG_ROWS):
    per_w = N // NW
    S = C * K
    nsuper = per_w // S
    assert nsuper % NBUF == 0
    mesh = plsc.VectorSubcoreMesh(core_axis_name="c", subcore_axis_name="s")

    @functools.partial(
        pl.kernel,
        out_type=jax.ShapeDtypeStruct((N, V_DIM), jnp.float32),
        mesh=mesh,
        scratch_types=[
            pltpu.VMEM((nchunk, C), jnp.int32),
            pltpu.VMEM((nchunk, C), jnp.int32),
            pltpu.VMEM((nchunk, C), jnp.int32),
            pltpu.VMEM((NBUF, S, V_DIM), jnp.float32),
            pltpu.VMEM((DEG_ROWS, V_DIM), jnp.float32),
            [pltpu.SemaphoreType.DMA] * NBUF,
            [pltpu.SemaphoreType.DMA] * NBUF,
        ],
        compiler_params=pltpu.CompilerParams(use_tc_tiling_on_sc=False),
    )
    def k(vidx_hbm, pos_hbm, deg_hbm, Wv, Wp, Wd, out_hbm,
          iv, ip, idg, rows, deg_tab, sa, sw):
        wid = lax.axis_index("s") * NC + lax.axis_index("c")
        base = wid * per_w
        pltpu.sync_copy(Wd, deg_tab)
        pltpu.sync_copy(vidx_hbm.at[wid], iv)
        pltpu.sync_copy(pos_hbm.at[wid], ip)
        pltpu.sync_copy(deg_hbm.at[wid], idg)

        def deg_fill(s, p):
            buf = rows.at[p]
            for t in range(K):
                c = s * K + t

                def grp(g, carry):
                    ixv = idg[c, pl.ds(g * 16, 16)]
                    for l in range(16):
                        ix = ixv[l]
                        r = t * C + g * 16 + l
                        buf[r, pl.ds(0, 16)] = deg_tab[ix, pl.ds(0, 16)]
                        buf[r, pl.ds(16, 16)] = deg_tab[ix, pl.ds(16, 16)]
                    return carry

                lax.fori_loop(0, C // 16, grp, 0)

        def fire_adds(s, p):
            buf = rows.at[p]
            for t in range(K):
                dst = buf.at[pl.ds(t * C, C)]
                pltpu.async_copy(Wv.at[iv.at[s * K + t]], dst, sa[p], add=True)
                pltpu.async_copy(Wp.at[ip.at[s * K + t]], dst, sa[p], add=True)

        def wait_adds(s, p):
            buf = rows.at[p]
            for t in range(K):
                dst = buf.at[pl.ds(t * C, C)]
                pltpu.make_async_copy(
                    Wv.at[iv.at[s * K + t]], dst, sa[p]).wait()
                pltpu.make_async_copy(
                    Wp.at[ip.at[s * K + t]], dst, sa[p]).wait()

        def fire_wb(s, p):
            pltpu.async_copy(
                rows.at[p], out_hbm.at[pl.ds(base + s * S, S)], sw[p])

        def wait_wb(p):
            pltpu.make_async_copy(
                rows.at[p], out_hbm.at[pl.ds(base, S)], sw[p]).wait()

        def step(s, u):
            p = u % NBUF
            q = (u + NBUF - 1) % NBUF
            @pl.when(s >= NBUF)
            def _():
                wait_wb(p)

            deg_fill(s, p)
            fire_adds(s, p)
            @pl.when(s >= 1)
            def _():
                wait_adds(s - 1, q)
                fire_wb(s - 1, q)

        def round_(g, carry):
            for u in range(NBUF):
                step(g * NBUF + u, u)
            return carry

        lax.fori_loop(0, nsuper // NBUF, round_, 0)
        p_last = (nsuper - 1) % NBUF
        wait_adds(nsuper - 1, p_last)
        fire_wb(nsuper - 1, p_last)
        for p in range(NBUF):
            wait_wb(p)

    return k


def kernel(vidx, pos, deg, W_vidx, W_pos, W_deg):
    B, L = vidx.shape
    N = B * L
    C = 128
    nchunk = N // (NW * C)
    iv = vidx.reshape(NW, nchunk, C).astype(jnp.int32)
    ip = pos.reshape(NW, nchunk, C).astype(jnp.int32)
    idg = deg.reshape(NW, nchunk, C).astype(jnp.int32)
    out = _make_sc_kernel(N, C, 2, nchunk, W_deg.shape[0])(
        iv, ip, idg, W_vidx, W_pos, W_deg)
    return out.reshape(B, L, V_DIM)
